# Initial kernel scaffold; baseline (speedup 1.0000x reference)
import jax
import jax.numpy as jnp
from jax.experimental import pallas as pl


def kernel(input_ids, word_table, oe_table, oe_projection):
    raise NotImplementedError("write your pallas kernel here")


# speedup vs baseline: 4.6521x; 4.6521x over previous
import jax
import jax.numpy as jnp
import numpy as np
from jax import lax
from jax.experimental import pallas as pl
from jax.experimental.pallas import tpu as pltpu
from jax.experimental.pallas import tpu_sc as plsc

T = 16384
D = 128
H = 16
G = 8
NUM_EMB = 100000
OE_M = 100003
OE_LINES = 100011

NC = 2
NS = 16
NW = NC * NS
TPW = T // NW
HALF = TPW // 2
PAD = 8

_EXCL = [0] * (G + 1)
for _i in range(G):
    _EXCL[_i + 1] = _EXCL[_i] + OE_M + 2 * _i + 1


def _fmod(x, m, inv):
    m = np.int32(m)
    q = (x.astype(jnp.float32) * inv).astype(jnp.int32)
    r = x - q * m
    r = jnp.where(r < np.int32(0), r + m, r)
    r = jnp.where(r >= m, r - m, r)
    return r


def _sc_body(padded_hbm, wt_hbm, oel_hbm, word_out, hid_out,
             toks_v, lidx_v, soff_v, widx_v, lines_v, hid_v, wrows_v, sem,
             wsem):
    cid = lax.axis_index("c")
    sid = lax.axis_index("s")
    wid = sid * np.int32(NC) + cid
    base = wid * np.int32(TPW)
    iota16 = lax.iota(jnp.int32, 16)
    for half in range(2):
        hbase = base + np.int32(half * HALF)
        pltpu.sync_copy(padded_hbm.at[pl.ds(hbase, HALF + PAD)], toks_v)

        for s in range(2):
            wrow = widx_v.at[s]

            def wbody(i, _, s=s, wrow=wrow):
                off = i * np.int32(16) + np.int32(s * 128 + PAD)
                wrow[pl.ds(i * np.int32(16), 16)] = toks_v[pl.ds(off, 16)]
                return np.int32(0)

            lax.fori_loop(np.int32(0), np.int32(8), wbody, np.int32(0))
        wdescs = [
            pltpu.make_async_copy(
                wt_hbm.at[widx_v.at[s]],
                wrows_v.at[pl.ds(s * 128, 128)], wsem)
            for s in range(2)
        ]
        for dsc in wdescs:
            dsc.start()

        for g in range(G):
            m = 100004 + 2 * g
            c = m - NUM_EMB
            inv = np.float32(1.0 / m)
            excl = _EXCL[g]
            for s in range(2):
                lrow = lidx_v.at[2 * g + s]
                orow = soff_v.at[2 * g + s]

                def gbody(i, _, s=s, m=m, c=c, inv=inv, excl=excl,
                          lrow=lrow, orow=orow, is3=(g >= 4)):
                    off = i * np.int32(16) + np.int32(s * 128 + PAD)
                    t0 = toks_v[pl.ds(off, 16)]
                    t1 = toks_v[pl.ds(off - np.int32(1), 16)]
                    r1 = _fmod(np.int32(c) * t1, m, inv)
                    acc = t0 - r1 + np.int32(m)
                    if is3:
                        t2 = toks_v[pl.ds(off - np.int32(2), 16)]
                        rr2 = _fmod(
                            np.int32(c) * _fmod(np.int32(c) * t2, m, inv),
                            m, inv)
                        acc = acc + rr2
                    gid = _fmod(acc, m, inv) + np.int32(excl)
                    j = i * np.int32(16)
                    lrow[pl.ds(j, 16)] = lax.shift_right_logical(
                        gid, np.int32(3))
                    orow[pl.ds(j, 16)] = lax.shift_left(
                        gid & np.int32(7), np.int32(4))
                    return np.int32(0)

                lax.fori_loop(np.int32(0), np.int32(8), gbody, np.int32(0))

        for g in range(G):
            ldescs = [
                pltpu.make_async_copy(
                    oel_hbm.at[lidx_v.at[2 * g + s]],
                    lines_v.at[pl.ds(s * 128, 128)], sem)
                for s in range(2)
            ]
            for dsc in ldescs:
                dsc.start()
            for dsc in ldescs:
                dsc.wait()
            for s in range(2):
                orow = soff_v.at[2 * g + s]

                def ebody(j, _, g=g, s=s, orow=orow):
                    rows = iota16 + (j * np.int32(16) + np.int32(s * 128))
                    soff = orow[pl.ds(j * np.int32(16), 16)]
                    for h in range(H):
                        vals = plsc.load_gather(
                            lines_v, [rows, soff + np.int32(h)])
                        plsc.store_scatter(
                            hid_v,
                            [rows, jnp.full((16,), g * H + h, jnp.int32)],
                            vals)
                    return np.int32(0)

                lax.fori_loop(np.int32(0), np.int32(8), ebody, np.int32(0))

        for dsc in wdescs:
            dsc.wait()
        pltpu.sync_copy(wrows_v, word_out.at[pl.ds(hbase, HALF)])
        pltpu.sync_copy(hid_v, hid_out.at[pl.ds(hbase, HALF)])


_sc_gather = pl.kernel(
    _sc_body,
    out_type=(
        jax.ShapeDtypeStruct((T, D), jnp.float32),
        jax.ShapeDtypeStruct((T, D), jnp.float32),
    ),
    mesh=plsc.VectorSubcoreMesh(
        core_axis_name="c", subcore_axis_name="s",
        num_cores=NC, num_subcores=NS),
    scratch_types=[
        pltpu.VMEM((HALF + PAD,), jnp.int32),
        pltpu.VMEM((2 * G, 128), jnp.int32),
        pltpu.VMEM((2 * G, 128), jnp.int32),
        pltpu.VMEM((2, 128), jnp.int32),
        pltpu.VMEM((HALF, D), jnp.float32),
        pltpu.VMEM((HALF, D), jnp.float32),
        pltpu.VMEM((HALF, D), jnp.float32),
        pltpu.SemaphoreType.DMA,
        pltpu.SemaphoreType.DMA,
    ],
    compiler_params=pltpu.CompilerParams(needs_layout_passes=False),
)


def _tc_body(word_ref, hid_ref, p_ref, out_ref):
    acc = jnp.dot(hid_ref[...], p_ref[...],
                  preferred_element_type=jnp.float32,
                  precision=lax.Precision.HIGHEST)
    out_ref[...] = (word_ref[...] + acc) * np.float32(1.0 / 9.0)


_TB = 2048


def _tc_project(word_emb, hidden, pfull):
    return pl.pallas_call(
        _tc_body,
        grid=(T // _TB,),
        in_specs=[
            pl.BlockSpec((_TB, D), lambda i: (i, 0)),
            pl.BlockSpec((_TB, D), lambda i: (i, 0)),
            pl.BlockSpec((D, D), lambda i: (0, 0)),
        ],
        out_specs=pl.BlockSpec((_TB, D), lambda i: (i, 0)),
        out_shape=jax.ShapeDtypeStruct((T, D), jnp.float32),
    )(word_emb, hidden, pfull)


def kernel(input_ids, word_table, oe_table, oe_projection):
    with jax.enable_x64(False):
        toks32 = input_ids.astype(jnp.int32)
        padded = jnp.concatenate([jnp.zeros((PAD,), jnp.int32), toks32])
        oe_lines = oe_table.reshape(OE_LINES, D)
        word_emb, hidden = _sc_gather(padded, word_table, oe_lines)
        pfull = oe_projection.reshape(D, D)
        return _tc_project(word_emb, hidden, pfull)
